# baseline (speedup 1.0000x reference)
import jax
import jax.numpy as jnp
from jax import lax
from jax.experimental import pallas as pl
from jax.experimental.pallas import tpu as pltpu
from jax.experimental.pallas import tpu_sc as plsc

B = 16
T = 2048
DIN = 256
HID = 256
K = 16
TT = 2048
NT = T // TT
BIG = 3.0e38
L = 16
BPB = 4


def _score_body(feat, para, pW1a, pW1b, pb1, pW2, pb2, fWb, fb,
                fWa, emb, Wc, A, P, sb1, sW2, g, out, pconst, cblk,
                wcat, pcA):
    t = pl.program_id(0)
    b = pl.program_id(1)

    @pl.when((t == 0) & (b == 0))
    def _():
        pv = para[...]
        h1 = (pv[:, 0:1] * pW1a[...][None, :]
              + pv[:, 1:2] * pW1b[...][None, :]
              + pb1[...][None, :])
        pe = (jnp.dot(jnp.maximum(h1, 0.0), pW2[...],
                      preferred_element_type=jnp.float32)
              + pb2[...][None, :])
        pc = (jnp.dot(pe, fWb[...], preferred_element_type=jnp.float32)
              + fb[...][None, :])
        pconst[...] = pc
        wcat[:, :HID] = fWa[...]
        wcat[:, HID:] = jnp.dot(fWa[...], A[...],
                                preferred_element_type=jnp.float32)
        pcA[...] = jnp.dot(pc, A[...], preferred_element_type=jnp.float32)

    @pl.when(b == 0)
    def _():
        cblk[...] = (jnp.dot(emb[...], Wc[...],
                             preferred_element_type=jnp.float32)
                     + sb1[...][None, :])

    w = wcat[...]
    Pm = P[...]
    sw = sW2[...]
    NCH = 4
    CH = TT // NCH
    for bi in range(BPB):
        bg = b * BPB + bi
        pc = pconst[pl.ds(bg, 1), :]
        pca = pcA[pl.ds(bg, 1), :]
        for c in range(NCH):
            x = feat[bi, pl.ds(c * CH, CH), :]
            ec = emb[pl.ds(c * CH, CH), :]
            r = jnp.dot(x, w, preferred_element_type=jnp.float32)
            fpm = r[:, :HID] + pc
            prod = fpm * ec
            pre = (jnp.dot(prod, Pm, preferred_element_type=jnp.float32)
                   + r[:, HID:] + pca + cblk[pl.ds(c * CH, CH), :])
            h = jnp.maximum(pre, 0.0)
            s = jnp.sum(h * sw[None, :], axis=1)
            out[bi, 0, pl.ds(c * CH, CH)] = s + g[bi, 0, pl.ds(c * CH, CH)]


def _scores(feat_seq, para, pW1a, pW1b, pb1, pW2, pb2, fWb, fb,
            fWa, emb_pad, Wc, A, P, sb1, sW2v, g3):
    return pl.pallas_call(
        _score_body,
        grid=(NT, B // BPB),
        in_specs=[
            pl.BlockSpec((BPB, TT, DIN), lambda t, b: (b, t, 0)),
            pl.BlockSpec((B, 2), lambda t, b: (0, 0)),
            pl.BlockSpec((2 * HID,), lambda t, b: (0,)),
            pl.BlockSpec((2 * HID,), lambda t, b: (0,)),
            pl.BlockSpec((2 * HID,), lambda t, b: (0,)),
            pl.BlockSpec((2 * HID, HID), lambda t, b: (0, 0)),
            pl.BlockSpec((HID,), lambda t, b: (0,)),
            pl.BlockSpec((HID, HID), lambda t, b: (0, 0)),
            pl.BlockSpec((HID,), lambda t, b: (0,)),
            pl.BlockSpec((DIN, HID), lambda t, b: (0, 0)),
            pl.BlockSpec((TT, HID), lambda t, b: (t, 0)),
            pl.BlockSpec((HID, HID), lambda t, b: (0, 0)),
            pl.BlockSpec((HID, HID), lambda t, b: (0, 0)),
            pl.BlockSpec((HID, HID), lambda t, b: (0, 0)),
            pl.BlockSpec((HID,), lambda t, b: (0,)),
            pl.BlockSpec((HID,), lambda t, b: (0,)),
            pl.BlockSpec((BPB, 1, TT), lambda t, b: (b, 0, 0)),
        ],
        out_specs=pl.BlockSpec((BPB, 1, TT), lambda t, b: (b, 0, 0)),
        out_shape=jax.ShapeDtypeStruct((B * NT, 1, TT), jnp.float32),
        scratch_shapes=[pltpu.VMEM((B, HID), jnp.float32),
                        pltpu.VMEM((TT, HID), jnp.float32),
                        pltpu.VMEM((DIN, 2 * HID), jnp.float32),
                        pltpu.VMEM((B, HID), jnp.float32)],
    )(feat_seq, para, pW1a, pW1b, pb1, pW2, pb2, fWb, fb,
      fWa, emb_pad, Wc, A, P, sb1, sW2v, g3)


def _topk_body(y_hbm, out_hbm, yv, ov):
    c = lax.axis_index("c")
    s = lax.axis_index("s")
    row = c * 16 + s

    @pl.when(row < B)
    def _():
        pltpu.sync_copy(y_hbm.at[row], yv)
        lanes = lax.broadcasted_iota(jnp.int32, (L,), 0)
        neg = jnp.full((L,), -BIG, jnp.float32)
        zero = jnp.zeros((L,), jnp.int32)

        def body(i, carry):
            tk0, ti0, tk1, ti1 = carry
            v0 = yv[pl.ds((2 * i) * L, L)]
            v1 = yv[pl.ds((2 * i + 1) * L, L)]
            a0k, a0i = plsc.sort_key_val(v0, lanes + (2 * i) * L,
                                         descending=True)
            a1k, a1i = plsc.sort_key_val(v1, lanes + (2 * i + 1) * L,
                                         descending=True)
            m0 = tk0 >= a0k
            m1 = tk1 >= a1k
            s0 = plsc.sort_key_val(jnp.where(m0, tk0, a0k),
                                   jnp.where(m0, ti0, a0i))
            s1 = plsc.sort_key_val(jnp.where(m1, tk1, a1k),
                                   jnp.where(m1, ti1, a1i))
            return (s0[0], s0[1], s1[0], s1[1])

        tk0, ti0, tk1, ti1 = lax.fori_loop(0, T // (2 * L), body,
                                           (neg, zero, neg, zero))
        r1k, r1i = plsc.sort_key_val(tk1, ti1, descending=True)
        m = tk0 >= r1k
        mi = jnp.where(m, ti0, r1i)
        si, _ = plsc.sort_key_val(mi, mi)
        ov[...] = si
        pltpu.sync_copy(ov, out_hbm.at[row])


def _sc_topk(y):
    mesh = plsc.VectorSubcoreMesh(core_axis_name="c", subcore_axis_name="s")
    kern = pl.kernel(
        _topk_body,
        mesh=mesh,
        out_type=jax.ShapeDtypeStruct((B, K), jnp.int32),
        scratch_types=[pltpu.VMEM((T,), jnp.float32),
                       pltpu.VMEM((K,), jnp.int32)],
        compiler_params=pltpu.CompilerParams(needs_layout_passes=False),
    )
    return kern(y)


def kernel(feat_seq, para, pW1, pb1, pW2, pb2, fW, fb, emb_table, sW1, sb1, sW2, sb2):
    pW1a = pW1[0]
    pW1b = pW1[1]
    fWa = fW[:DIN]
    fWb = fW[DIN:]
    A = sW1[0:HID] + sW1[2 * HID:3 * HID]
    Wc = sW1[HID:2 * HID] - sW1[2 * HID:3 * HID]
    P = sW1[3 * HID:4 * HID] + sW1[4 * HID:4 * HID + 1]
    emb_pad = jnp.pad(emb_table, ((1, 1), (0, 0)))
    g = jax.random.gumbel(jax.random.key(42), (B, T - 2), jnp.float32)
    g3 = jnp.pad(g + sb2[0], ((0, 0), (1, 1)),
                 constant_values=BIG).reshape(B * NT, 1, TT)
    sW2v = sW2[:, 0]

    y3 = _scores(feat_seq, para, pW1a, pW1b, pb1, pW2, pb2, fWb, fb,
                 fWa, emb_pad, Wc, A, P, sb1, sW2v, g3)
    return _sc_topk(y3.reshape(B, T))


# speedup vs baseline: 1.0337x; 1.0337x over previous
import jax
import jax.numpy as jnp
from jax import lax
from jax.experimental import pallas as pl
from jax.experimental.pallas import tpu as pltpu
from jax.experimental.pallas import tpu_sc as plsc

B = 16
T = 2048
DIN = 256
HID = 256
K = 16
TT = 2048
NT = T // TT
BIG = 3.0e38
L = 16
BPB = 4


def _score_body(feat, para, pW1a, pW1b, pb1, pW2, pb2, fWb, fb,
                fWa, emb, Wc, A, P, sb1, sW2, g, out, pconst, cblk,
                wcat, pcA):
    t = pl.program_id(0)
    b = pl.program_id(1)

    @pl.when((t == 0) & (b == 0))
    def _():
        pv = para[...]
        h1 = (pv[:, 0:1] * pW1a[...][None, :]
              + pv[:, 1:2] * pW1b[...][None, :]
              + pb1[...][None, :])
        pe = (jnp.dot(jnp.maximum(h1, 0.0), pW2[...],
                      preferred_element_type=jnp.float32)
              + pb2[...][None, :])
        pc = (jnp.dot(pe, fWb[...], preferred_element_type=jnp.float32)
              + fb[...][None, :])
        pconst[...] = pc
        wcat[:, :HID] = fWa[...]
        wcat[:, HID:] = jnp.dot(fWa[...], A[...],
                                preferred_element_type=jnp.float32)
        pcA[...] = jnp.dot(pc, A[...], preferred_element_type=jnp.float32)

    @pl.when(b == 0)
    def _():
        cblk[...] = (jnp.dot(emb[...], Wc[...],
                             preferred_element_type=jnp.float32)
                     + sb1[...][None, :])

    w = wcat[...]
    Pm = P[...]
    sw = sW2[...]
    NCH = 8
    CH = TT // NCH
    for bi in range(BPB):
        bg = b * BPB + bi
        pc = pconst[pl.ds(bg, 1), :]
        pca = pcA[pl.ds(bg, 1), :]
        for c in range(NCH):
            x = feat[bi, pl.ds(c * CH, CH), :]
            ec = emb[pl.ds(c * CH, CH), :]
            r = jnp.dot(x, w, preferred_element_type=jnp.float32)
            fpm = r[:, :HID] + pc
            prod = fpm * ec
            pre = (jnp.dot(prod, Pm, preferred_element_type=jnp.float32)
                   + r[:, HID:] + pca + cblk[pl.ds(c * CH, CH), :])
            h = jnp.maximum(pre, 0.0)
            s = jnp.sum(h * sw[None, :], axis=1)
            out[bi, 0, pl.ds(c * CH, CH)] = s + g[bi, 0, pl.ds(c * CH, CH)]


def _scores(feat_seq, para, pW1a, pW1b, pb1, pW2, pb2, fWb, fb,
            fWa, emb_pad, Wc, A, P, sb1, sW2v, g3):
    return pl.pallas_call(
        _score_body,
        grid=(NT, B // BPB),
        in_specs=[
            pl.BlockSpec((BPB, TT, DIN), lambda t, b: (b, t, 0)),
            pl.BlockSpec((B, 2), lambda t, b: (0, 0)),
            pl.BlockSpec((2 * HID,), lambda t, b: (0,)),
            pl.BlockSpec((2 * HID,), lambda t, b: (0,)),
            pl.BlockSpec((2 * HID,), lambda t, b: (0,)),
            pl.BlockSpec((2 * HID, HID), lambda t, b: (0, 0)),
            pl.BlockSpec((HID,), lambda t, b: (0,)),
            pl.BlockSpec((HID, HID), lambda t, b: (0, 0)),
            pl.BlockSpec((HID,), lambda t, b: (0,)),
            pl.BlockSpec((DIN, HID), lambda t, b: (0, 0)),
            pl.BlockSpec((TT, HID), lambda t, b: (t, 0)),
            pl.BlockSpec((HID, HID), lambda t, b: (0, 0)),
            pl.BlockSpec((HID, HID), lambda t, b: (0, 0)),
            pl.BlockSpec((HID, HID), lambda t, b: (0, 0)),
            pl.BlockSpec((HID,), lambda t, b: (0,)),
            pl.BlockSpec((HID,), lambda t, b: (0,)),
            pl.BlockSpec((BPB, 1, TT), lambda t, b: (b, 0, 0)),
        ],
        out_specs=pl.BlockSpec((BPB, 1, TT), lambda t, b: (b, 0, 0)),
        out_shape=jax.ShapeDtypeStruct((B * NT, 1, TT), jnp.float32),
        scratch_shapes=[pltpu.VMEM((B, HID), jnp.float32),
                        pltpu.VMEM((TT, HID), jnp.float32),
                        pltpu.VMEM((DIN, 2 * HID), jnp.float32),
                        pltpu.VMEM((B, HID), jnp.float32)],
    )(feat_seq, para, pW1a, pW1b, pb1, pW2, pb2, fWb, fb,
      fWa, emb_pad, Wc, A, P, sb1, sW2v, g3)


def _topk_body(y_hbm, out_hbm, yv, ov):
    c = lax.axis_index("c")
    s = lax.axis_index("s")
    row = c * 16 + s

    @pl.when(row < B)
    def _():
        pltpu.sync_copy(y_hbm.at[row], yv)
        lanes = lax.broadcasted_iota(jnp.int32, (L,), 0)
        neg = jnp.full((L,), -BIG, jnp.float32)
        zero = jnp.zeros((L,), jnp.int32)

        def body(i, carry):
            tk0, ti0, tk1, ti1 = carry
            v0 = yv[pl.ds((2 * i) * L, L)]
            v1 = yv[pl.ds((2 * i + 1) * L, L)]
            a0k, a0i = plsc.sort_key_val(v0, lanes + (2 * i) * L,
                                         descending=True)
            a1k, a1i = plsc.sort_key_val(v1, lanes + (2 * i + 1) * L,
                                         descending=True)
            m0 = tk0 >= a0k
            m1 = tk1 >= a1k
            s0 = plsc.sort_key_val(jnp.where(m0, tk0, a0k),
                                   jnp.where(m0, ti0, a0i))
            s1 = plsc.sort_key_val(jnp.where(m1, tk1, a1k),
                                   jnp.where(m1, ti1, a1i))
            return (s0[0], s0[1], s1[0], s1[1])

        tk0, ti0, tk1, ti1 = lax.fori_loop(0, T // (2 * L), body,
                                           (neg, zero, neg, zero))
        r1k, r1i = plsc.sort_key_val(tk1, ti1, descending=True)
        m = tk0 >= r1k
        mi = jnp.where(m, ti0, r1i)
        si, _ = plsc.sort_key_val(mi, mi)
        ov[...] = si
        pltpu.sync_copy(ov, out_hbm.at[row])


def _sc_topk(y):
    mesh = plsc.VectorSubcoreMesh(core_axis_name="c", subcore_axis_name="s")
    kern = pl.kernel(
        _topk_body,
        mesh=mesh,
        out_type=jax.ShapeDtypeStruct((B, K), jnp.int32),
        scratch_types=[pltpu.VMEM((T,), jnp.float32),
                       pltpu.VMEM((K,), jnp.int32)],
        compiler_params=pltpu.CompilerParams(needs_layout_passes=False),
    )
    return kern(y)


def kernel(feat_seq, para, pW1, pb1, pW2, pb2, fW, fb, emb_table, sW1, sb1, sW2, sb2):
    pW1a = pW1[0]
    pW1b = pW1[1]
    fWa = fW[:DIN]
    fWb = fW[DIN:]
    A = sW1[0:HID] + sW1[2 * HID:3 * HID]
    Wc = sW1[HID:2 * HID] - sW1[2 * HID:3 * HID]
    P = sW1[3 * HID:4 * HID] + sW1[4 * HID:4 * HID + 1]
    emb_pad = jnp.pad(emb_table, ((1, 1), (0, 0)))
    g = jax.random.gumbel(jax.random.key(42), (B, T - 2), jnp.float32)
    g3 = jnp.pad(g + sb2[0], ((0, 0), (1, 1)),
                 constant_values=BIG).reshape(B * NT, 1, TT)
    sW2v = sW2[:, 0]

    y3 = _scores(feat_seq, para, pW1a, pW1b, pb1, pW2, pb2, fWb, fb,
                 fWa, emb_pad, Wc, A, P, sb1, sW2v, g3)
    return _sc_topk(y3.reshape(B, T))
